# baseline (speedup 1.0000x reference)
import functools

import jax
import jax.numpy as jnp
from jax import lax
from jax.experimental import pallas as pl
from jax.experimental.pallas import tpu as pltpu
from jax.experimental.pallas import tpu_sc as plsc

N = 10000
E = 320000
D = 128
H = 128
O = 128
NLAYER = 3
ALPHA = 0.6

NC = 2
NS = 16
NW = NC * NS
CH = 128
CPW = 79
EPAD = NW * CPW * CH
NPAD = 10112
ZPT = NPAD // NS
MBLK = 1000
GRID = N // MBLK

_mesh = plsc.VectorSubcoreMesh(
    core_axis_name="c", subcore_axis_name="s", num_cores=NC, num_subcores=NS)


@functools.partial(
    pl.kernel,
    out_type=jax.ShapeDtypeStruct((NC, NPAD, H), jnp.float32),
    mesh=_mesh,
    scratch_types=[
        pltpu.VMEM((CH,), jnp.int32),
        pltpu.VMEM((CH,), jnp.int32),
        pltpu.VMEM((CH, H), jnp.float32),
        pltpu.VMEM_SHARED((NPAD, H), jnp.float32),
        pltpu.SemaphoreType.DMA,
    ],
)
def _scatter_kernel(hp_hbm, src_hbm, dst_hbm, out_hbm,
                    srcv, dstv, rows, acc_sh, sem):
    cc = lax.axis_index("c")
    s = lax.axis_index("s")

    def fill_zero(i, carry):
        for k in range(H // 16):
            rows[i, pl.ds(k * 16, 16)] = jnp.zeros((16,), jnp.float32)
        return carry
    lax.fori_loop(0, CH, fill_zero, 0)

    for j in range(ZPT // CH):
        pltpu.sync_copy(rows, acc_sh.at[pl.ds(s * ZPT + j * CH, CH)])
    rem = ZPT % CH
    if rem:
        pltpu.sync_copy(rows.at[pl.ds(0, rem)],
                        acc_sh.at[pl.ds(s * ZPT + (ZPT // CH) * CH, rem)])
    plsc.subcore_barrier()

    wid = s * NC + cc

    def chunk(j, carry):
        off = (wid * CPW + j) * CH
        pltpu.sync_copy(src_hbm.at[pl.ds(off, CH)], srcv)
        pltpu.sync_copy(dst_hbm.at[pl.ds(off, CH)], dstv)
        pltpu.async_copy(hp_hbm.at[srcv], rows, sem).wait()
        pltpu.sync_copy(rows, acc_sh.at[dstv], add=True)
        return carry
    lax.fori_loop(0, CPW, chunk, 0)

    plsc.subcore_barrier()
    pltpu.sync_copy(acc_sh.at[pl.ds(s * ZPT, ZPT)],
                    out_hbm.at[cc, pl.ds(s * ZPT, ZPT)])


def _dot_t(a, w):
    return lax.dot_general(a, w, (((1,), (1,)), ((), ())),
                           preferred_element_type=jnp.float32)


def _tc_pre_body(x_ref, w1_ref, b1_ref, zinit_ref):
    zinit_ref[...] = _dot_t(x_ref[...], w1_ref[...]) + b1_ref[...]


def _tc_deg_body(s_ref, disb_ref):
    disb_ref[...] = lax.rsqrt(s_ref[0] + s_ref[1] + 1.0)


def _tc_hp_body(z_ref, wc_ref, disb_ref, hp_ref):
    hp_ref[...] = _dot_t(z_ref[...], wc_ref[...]) * disb_ref[...]


def _tc_layer_body(s_ref, hp_ref, disb_ref, bc_ref, accin_ref,
                   z_ref, acc_ref):
    z = jnp.maximum(
        disb_ref[...] * (s_ref[0] + s_ref[1] + hp_ref[...]) + bc_ref[...],
        0.0)
    z_ref[...] = z
    acc_ref[...] = accin_ref[...] + z


def _tc_post_body(zinit_ref, acc_ref, w2_ref, b2_ref, out_ref):
    ox = ALPHA * zinit_ref[...] + (1.0 - ALPHA) / NLAYER * acc_ref[...]
    out_ref[...] = _dot_t(ox, w2_ref[...]) + b2_ref[...]


def _row_spec():
    return pl.BlockSpec((MBLK, H), lambda i: (i, 0))


def _full_spec(shape):
    nd = len(shape)
    return pl.BlockSpec(shape, lambda i, _n=nd: (0,) * _n)


def _s_spec():
    return pl.BlockSpec((NC, MBLK, H), lambda i: (0, i, 0))


_f32 = jnp.float32
_row_sds = jax.ShapeDtypeStruct((N, H), _f32)

_tc_pre = pl.pallas_call(
    _tc_pre_body,
    grid=(GRID,),
    in_specs=[_row_spec(), _full_spec((H, D)), _full_spec((1, H))],
    out_specs=_row_spec(),
    out_shape=_row_sds,
)

_tc_deg = pl.pallas_call(
    _tc_deg_body,
    grid=(GRID,),
    in_specs=[_s_spec()],
    out_specs=_row_spec(),
    out_shape=_row_sds,
)

_tc_hp = pl.pallas_call(
    _tc_hp_body,
    grid=(GRID,),
    in_specs=[_row_spec(), _full_spec((H, H)), _row_spec()],
    out_specs=_row_spec(),
    out_shape=_row_sds,
)

_tc_layer = pl.pallas_call(
    _tc_layer_body,
    grid=(GRID,),
    in_specs=[_s_spec(), _row_spec(), _row_spec(), _full_spec((1, H)),
              _row_spec()],
    out_specs=[_row_spec(), _row_spec()],
    out_shape=[_row_sds, _row_sds],
)

_tc_post = pl.pallas_call(
    _tc_post_body,
    grid=(GRID,),
    in_specs=[_row_spec(), _row_spec(), _full_spec((O, H)),
              _full_spec((1, O))],
    out_specs=_row_spec(),
    out_shape=jax.ShapeDtypeStruct((N, O), _f32),
)


@jax.jit
def kernel(x, edge_index, W1, b1, Wc, bc, W2, b2):
    src, dst = edge_index[0], edge_index[1]
    pad = EPAD - E
    srcp = jnp.concatenate([src, jnp.zeros((pad,), src.dtype)])
    dstp = jnp.concatenate([dst, jnp.full((pad,), N, dst.dtype)])

    zinit = _tc_pre(x, W1, b1.reshape(1, H))
    hp_ones = jnp.ones((N, H), _f32)

    def pass_fn(carry, xs):
        z, acc, disb = carry
        i, wc_i, bc_i = xs
        hp = lax.cond(i == 0, lambda: hp_ones,
                      lambda: _tc_hp(z, wc_i, disb))
        src_i = jnp.where(i == 0, 0, srcp)
        s_part = _scatter_kernel(hp, src_i, dstp)

        def first():
            return z, acc, _tc_deg(s_part)

        def rest():
            z2, acc2 = _tc_layer(s_part, hp, disb, bc_i, acc)
            return z2, acc2, disb

        return lax.cond(i == 0, first, rest), None

    wc_x = jnp.concatenate([Wc[:1], Wc])
    bc_x = jnp.concatenate([bc[:1], bc]).reshape(NLAYER + 1, 1, H)
    (_, acc, _), _ = lax.scan(
        pass_fn,
        (zinit, jnp.zeros((N, H), _f32), jnp.ones((N, H), _f32)),
        (jnp.arange(NLAYER + 1), wc_x, bc_x))

    return _tc_post(zinit, acc, W2, b2.reshape(1, O))


# speedup vs baseline: 7.3719x; 7.3719x over previous
import functools

import jax
import jax.numpy as jnp
from jax import lax
from jax.experimental import pallas as pl
from jax.experimental.pallas import tpu as pltpu
from jax.experimental.pallas import tpu_sc as plsc

N = 10000
E = 320000
D = 128
H = 128
O = 128
NLAYER = 3
ALPHA = 0.6

NC = 2
NS = 16
NW = NC * NS
CH = 128
CPW = 79
EPAD = NW * CPW * CH
NPAD = 10112
ZPT = NPAD // NS
MBLK = 1000
GRID = N // MBLK

_mesh = plsc.VectorSubcoreMesh(
    core_axis_name="c", subcore_axis_name="s", num_cores=NC, num_subcores=NS)


@functools.partial(
    pl.kernel,
    out_type=jax.ShapeDtypeStruct((NC, NPAD, H), jnp.float32),
    mesh=_mesh,
    scratch_types=[
        pltpu.VMEM((CH,), jnp.int32),
        pltpu.VMEM((CH,), jnp.int32),
        pltpu.VMEM((CH, H), jnp.float32),
        pltpu.VMEM_SHARED((NPAD, H), jnp.float32),
        pltpu.SemaphoreType.DMA,
    ],
)
def _scatter_kernel(hp_hbm, src_hbm, dst_hbm, out_hbm,
                    srcv, dstv, rows, acc_sh, sem):
    cc = lax.axis_index("c")
    s = lax.axis_index("s")

    def fill_zero(i, carry):
        for k in range(H // 16):
            rows[i, pl.ds(k * 16, 16)] = jnp.zeros((16,), jnp.float32)
        return carry
    lax.fori_loop(0, CH, fill_zero, 0)

    for j in range(ZPT // CH):
        pltpu.sync_copy(rows, acc_sh.at[pl.ds(s * ZPT + j * CH, CH)])
    rem = ZPT % CH
    if rem:
        pltpu.sync_copy(rows.at[pl.ds(0, rem)],
                        acc_sh.at[pl.ds(s * ZPT + (ZPT // CH) * CH, rem)])
    plsc.subcore_barrier()

    wid = s * NC + cc

    def chunk(j, carry):
        off = (wid * CPW + j) * CH
        pltpu.sync_copy(src_hbm.at[pl.ds(off, CH)], srcv)
        pltpu.sync_copy(dst_hbm.at[pl.ds(off, CH)], dstv)
        pltpu.async_copy(hp_hbm.at[srcv], rows, sem).wait()
        pltpu.sync_copy(rows, acc_sh.at[dstv], add=True)
        return carry
    lax.fori_loop(0, CPW, chunk, 0)

    plsc.subcore_barrier()
    pltpu.sync_copy(acc_sh.at[pl.ds(s * ZPT, ZPT)],
                    out_hbm.at[cc, pl.ds(s * ZPT, ZPT)])


def _dot_t(a, w):
    return lax.dot_general(a, w, (((1,), (1,)), ((), ())),
                           preferred_element_type=jnp.float32)


def _tc_pre_body(x_ref, w1_ref, b1_ref, zinit_ref):
    zinit_ref[...] = _dot_t(x_ref[...], w1_ref[...]) + b1_ref[...]


def _tc_deg_body(s_ref, disb_ref):
    disb_ref[...] = lax.rsqrt(s_ref[0] + s_ref[1] + 1.0)


def _tc_hp_body(z_ref, wc_ref, disb_ref, hp_ref):
    hp_ref[...] = _dot_t(z_ref[...], wc_ref[...]) * disb_ref[...]


def _tc_layer_body(s_ref, hp_ref, disb_ref, bc_ref, accin_ref,
                   z_ref, acc_ref):
    z = jnp.maximum(
        disb_ref[...] * (s_ref[0] + s_ref[1] + hp_ref[...]) + bc_ref[...],
        0.0)
    z_ref[...] = z
    acc_ref[...] = accin_ref[...] + z


def _tc_post_body(zinit_ref, acc_ref, w2_ref, b2_ref, out_ref):
    ox = ALPHA * zinit_ref[...] + (1.0 - ALPHA) / NLAYER * acc_ref[...]
    out_ref[...] = _dot_t(ox, w2_ref[...]) + b2_ref[...]


def _row_spec():
    return pl.BlockSpec((MBLK, H), lambda i: (i, 0))


def _full_spec(shape):
    nd = len(shape)
    return pl.BlockSpec(shape, lambda i, _n=nd: (0,) * _n)


def _s_spec():
    return pl.BlockSpec((NC, MBLK, H), lambda i: (0, i, 0))


_f32 = jnp.float32
_row_sds = jax.ShapeDtypeStruct((N, H), _f32)

_tc_pre = pl.pallas_call(
    _tc_pre_body,
    grid=(GRID,),
    in_specs=[_row_spec(), _full_spec((H, D)), _full_spec((1, H))],
    out_specs=_row_spec(),
    out_shape=_row_sds,
)

_tc_deg = pl.pallas_call(
    _tc_deg_body,
    grid=(GRID,),
    in_specs=[_s_spec()],
    out_specs=_row_spec(),
    out_shape=_row_sds,
)

_tc_hp = pl.pallas_call(
    _tc_hp_body,
    grid=(GRID,),
    in_specs=[_row_spec(), _full_spec((H, H)), _row_spec()],
    out_specs=_row_spec(),
    out_shape=_row_sds,
)

_tc_layer = pl.pallas_call(
    _tc_layer_body,
    grid=(GRID,),
    in_specs=[_s_spec(), _row_spec(), _row_spec(), _full_spec((1, H)),
              _row_spec()],
    out_specs=[_row_spec(), _row_spec()],
    out_shape=[_row_sds, _row_sds],
)

_tc_post = pl.pallas_call(
    _tc_post_body,
    grid=(GRID,),
    in_specs=[_row_spec(), _row_spec(), _full_spec((O, H)),
              _full_spec((1, O))],
    out_specs=_row_spec(),
    out_shape=jax.ShapeDtypeStruct((N, O), _f32),
)


@jax.jit
def kernel(x, edge_index, W1, b1, Wc, bc, W2, b2):
    src, dst = edge_index[0], edge_index[1]
    pad = EPAD - E
    srcp = jnp.concatenate([src, jnp.zeros((pad,), src.dtype)])
    dstp = jnp.concatenate([dst, jnp.full((pad,), N, dst.dtype)])

    zinit = _tc_pre(x, W1, b1.reshape(1, H))
    hp_ones = jnp.ones((N, H), _f32)

    def pass_fn(carry, xs):
        z, acc, disb = carry
        i, wc_i, bc_i = xs
        hp = lax.cond(i == 0, lambda: hp_ones,
                      lambda: _tc_hp(z, wc_i, disb))
        s_part = _scatter_kernel(hp, srcp, dstp)

        def first():
            return z, acc, _tc_deg(s_part)

        def rest():
            z2, acc2 = _tc_layer(s_part, hp, disb, bc_i, acc)
            return z2, acc2, disb

        return lax.cond(i == 0, first, rest), None

    wc_x = jnp.concatenate([Wc[:1], Wc])
    bc_x = jnp.concatenate([bc[:1], bc]).reshape(NLAYER + 1, 1, H)
    (_, acc, _), _ = lax.scan(
        pass_fn,
        (zinit, jnp.zeros((N, H), _f32), jnp.ones((N, H), _f32)),
        (jnp.arange(NLAYER + 1), wc_x, bc_x))

    return _tc_post(zinit, acc, W2, b2.reshape(1, O))
